# baseline (speedup 1.0000x reference)
import functools

import jax
import jax.numpy as jnp
from jax.experimental import pallas as pl
from jax.experimental.pallas import tpu as pltpu

_NUM_SEL = 256


def _qsa_kernel(pt_ref, w1_ref, b1_ref, w2_ref, b2_ref,
                emb_ref, amap_ref):
    pt = pt_ref[0]
    p = jnp.transpose(pt)
    n = p.shape[0]

    dots = jnp.dot(p, pt, preferred_element_type=jnp.float32)
    m = jnp.max(dots, axis=1, keepdims=True)
    t = dots - m
    e = jnp.exp(t)
    s = jnp.sum(e, axis=1, keepdims=True)

    ls = jnp.log(s)
    u = jnp.sum(e * t, axis=1, keepdims=True)
    ent = ls - u / s

    ii = jax.lax.broadcasted_iota(jnp.int32, (n, n), 0)
    jj = jax.lax.broadcasted_iota(jnp.int32, (n, n), 1)

    ent_row = jnp.transpose(ent)
    less = ent_row < ent
    tie = (ent_row == ent) & (jj < ii)
    cmp = jnp.where(less | tie, 1, 0).astype(jnp.int32)
    rank = jnp.sum(cmp, axis=1, keepdims=True)

    rank_row = jnp.transpose(rank)
    kk = jax.lax.broadcasted_iota(jnp.int32, (_NUM_SEL, n), 0)
    st = jnp.where(rank_row == kk, 1.0, 0.0)

    e_sel = jnp.dot(st, e,
                    preferred_element_type=jnp.float32)
    s_sel = jnp.sum(e_sel, axis=1, keepdims=True)
    amap = e_sel / s_sel
    amap_ref[0] = amap

    sampled = jnp.dot(amap, p,
                      preferred_element_type=jnp.float32)
    h = jnp.maximum(
        jnp.dot(sampled, w1_ref[...],
                preferred_element_type=jnp.float32) + b1_ref[...], 0.0)
    emb = jnp.dot(h, w2_ref[...],
                  preferred_element_type=jnp.float32) + b2_ref[...]
    nrm = jnp.sqrt(jnp.sum(emb * emb, axis=1, keepdims=True))
    emb_ref[0] = emb / jnp.maximum(nrm, 1e-12)


@functools.partial(jax.jit, static_argnames=())
def kernel(layer_outs, W1, b1, W2, b2):
    B, C, H, Wd = layer_outs.shape
    N = H * Wd
    E = W2.shape[1]
    pt = layer_outs.reshape(B, C, N)
    b1r = b1.reshape(1, E)
    b2r = b2.reshape(1, E)

    emb, amap = pl.pallas_call(
        _qsa_kernel,
        grid=(B,),
        in_specs=[
            pl.BlockSpec((1, C, N), lambda b: (b, 0, 0)),
            pl.BlockSpec((C, E), lambda b: (0, 0)),
            pl.BlockSpec((1, E), lambda b: (0, 0)),
            pl.BlockSpec((E, E), lambda b: (0, 0)),
            pl.BlockSpec((1, E), lambda b: (0, 0)),
        ],
        out_specs=[
            pl.BlockSpec((1, _NUM_SEL, E), lambda b: (b, 0, 0)),
            pl.BlockSpec((1, _NUM_SEL, N), lambda b: (b, 0, 0)),
        ],
        out_shape=[
            jax.ShapeDtypeStruct((B, _NUM_SEL, E), jnp.float32),
            jax.ShapeDtypeStruct((B, _NUM_SEL, N), jnp.float32),
        ],
        compiler_params=pltpu.CompilerParams(
            dimension_semantics=("arbitrary",)),
    )(pt, W1, b1r, W2, b2r)
    return (emb, amap)


# speedup vs baseline: 1.0105x; 1.0105x over previous
import functools

import jax
import jax.numpy as jnp
from jax.experimental import pallas as pl
from jax.experimental.pallas import tpu as pltpu

_NUM_SEL = 256


def _qsa_kernel(pt_ref, w1_ref, b1_ref, w2_ref, b2_ref,
                emb_ref, amap_ref):
  for _b in range(pt_ref.shape[0]):
      pt = pt_ref[_b]
      p = jnp.transpose(pt)
      n = p.shape[0]

      dots = jnp.dot(p, pt, preferred_element_type=jnp.float32)
      m = jnp.max(dots, axis=1, keepdims=True)
      t = dots - m
      e = jnp.exp(t)
      s = jnp.sum(e, axis=1, keepdims=True)

      ls = jnp.log(s)
      u = jnp.sum(e * t, axis=1, keepdims=True)
      ent = ls - u / s

      ii = jax.lax.broadcasted_iota(jnp.int32, (n, n), 0)
      jj = jax.lax.broadcasted_iota(jnp.int32, (n, n), 1)

      ent_row = jnp.transpose(ent)
      less = ent_row < ent
      tie = (ent_row == ent) & (jj < ii)
      cmp = jnp.where(less | tie, 1, 0).astype(jnp.int32)
      rank = jnp.sum(cmp, axis=1, keepdims=True)

      rank_row = jnp.transpose(rank)
      kk = jax.lax.broadcasted_iota(jnp.int32, (_NUM_SEL, n), 0)
      st = jnp.where(rank_row == kk, 1.0, 0.0)

      e_sel = jnp.dot(st, e,
                      preferred_element_type=jnp.float32)
      s_sel = jnp.sum(e_sel, axis=1, keepdims=True)
      amap = e_sel / s_sel
      amap_ref[_b] = amap

      sampled = jnp.dot(amap, p,
                        preferred_element_type=jnp.float32)
      h = jnp.maximum(
          jnp.dot(sampled, w1_ref[...],
                  preferred_element_type=jnp.float32) + b1_ref[...], 0.0)
      emb = jnp.dot(h, w2_ref[...],
                    preferred_element_type=jnp.float32) + b2_ref[...]
      nrm = jnp.sqrt(jnp.sum(emb * emb, axis=1, keepdims=True))
      emb_ref[_b] = emb / jnp.maximum(nrm, 1e-12)


@functools.partial(jax.jit, static_argnames=())
def kernel(layer_outs, W1, b1, W2, b2):
    B, C, H, Wd = layer_outs.shape
    N = H * Wd
    E = W2.shape[1]
    pt = layer_outs.reshape(B, C, N)
    b1r = b1.reshape(1, E)
    b2r = b2.reshape(1, E)

    emb, amap = pl.pallas_call(
        _qsa_kernel,
        grid=(B // 2,),
        in_specs=[
            pl.BlockSpec((2, C, N), lambda b: (b, 0, 0)),
            pl.BlockSpec((C, E), lambda b: (0, 0)),
            pl.BlockSpec((1, E), lambda b: (0, 0)),
            pl.BlockSpec((E, E), lambda b: (0, 0)),
            pl.BlockSpec((1, E), lambda b: (0, 0)),
        ],
        out_specs=[
            pl.BlockSpec((2, _NUM_SEL, E), lambda b: (b, 0, 0)),
            pl.BlockSpec((2, _NUM_SEL, N), lambda b: (b, 0, 0)),
        ],
        out_shape=[
            jax.ShapeDtypeStruct((B, _NUM_SEL, E), jnp.float32),
            jax.ShapeDtypeStruct((B, _NUM_SEL, N), jnp.float32),
        ],
        compiler_params=pltpu.CompilerParams(
            dimension_semantics=("arbitrary",)),
    )(pt, W1, b1r, W2, b2r)
    return (emb, amap)
